# baseline (speedup 1.0000x reference)
import functools

import jax
import jax.numpy as jnp
from jax import lax
from jax.experimental import pallas as pl
from jax.experimental.pallas import tpu as pltpu
from jax.experimental.pallas import tpu_sc as plsc

V, D, H, FF, L, E, TOPK = 32000, 768, 12, 2048, 2, 8, 2
B, S = 1, 2048
DH = D // H

_EPS = 1e-5

_NA = S * TOPK
_BMOE = 512
_NB = _NA // _BMOE + E
_NBPAD = 32
_NSLOT = _NB * _BMOE
_CAP = S


def _ln_rows(y, g, b):
    m = jnp.mean(y, axis=-1, keepdims=True)
    v = jnp.mean((y - m) ** 2, axis=-1, keepdims=True)
    return (y - m) * lax.rsqrt(v + _EPS) * g + b


_GROWS = 8


def _gather_kernel(*refs):
    out_ref = refs[-1]
    for j in range(_GROWS):
        out_ref[j, :] = refs[1 + j][0, 0, :]


def _gather_rows(table, idx, n_rows):
    t3 = table.reshape(table.shape[0], 1, D)
    grid_spec = pltpu.PrefetchScalarGridSpec(
        num_scalar_prefetch=1,
        grid=(n_rows // _GROWS,),
        in_specs=[
            pl.BlockSpec((1, 1, D), functools.partial(
                lambda j, i, ids: (ids[i * _GROWS + j], 0, 0), j))
            for j in range(_GROWS)
        ],
        out_specs=pl.BlockSpec((_GROWS, D), lambda i, ids: (i, 0)),
    )
    return pl.pallas_call(
        _gather_kernel,
        grid_spec=grid_spec,
        out_shape=jax.ShapeDtypeStruct((n_rows, D), jnp.float32),
    )(idx, *([t3] * _GROWS))


_NW = 32
_GCH = 64


def _sc_gather(table, idx, n_rows):
    per = n_rows // _NW

    def body(table_hbm, idx_hbm, out_hbm, idx_v, rows_v, sem):
        wid = lax.axis_index("s") * 2 + lax.axis_index("c")
        base = wid * per
        for t in range(per // _GCH):
            off = pl.multiple_of(base + t * _GCH, _GCH)
            pltpu.sync_copy(idx_hbm.at[pl.ds(off, _GCH)], idx_v)
            pltpu.async_copy(table_hbm.at[idx_v], rows_v, sem).wait()
            pltpu.sync_copy(rows_v, out_hbm.at[pl.ds(off, _GCH)])

    k = functools.partial(
        pl.kernel,
        mesh=plsc.VectorSubcoreMesh(core_axis_name="c", subcore_axis_name="s"),
        out_type=jax.ShapeDtypeStruct((n_rows, D), jnp.float32),
        scratch_types=[
            pltpu.VMEM((_GCH,), jnp.int32),
            pltpu.VMEM((_GCH, D), jnp.float32),
            pltpu.SemaphoreType.DMA,
        ],
    )(body)
    return k(table, idx)


def _qkv_kernel(x_ref, w_ref, b_ref, o_ref):
    y = lax.dot_general(x_ref[...].astype(jnp.bfloat16), w_ref[...],
                        (((1,), (1,)), ((), ())),
                        preferred_element_type=jnp.float32) + b_ref[0]
    o_ref[...] = y[None]


def _qkv_proj(x, Wqkv, bqkv):
    BM = 512
    return pl.pallas_call(
        _qkv_kernel,
        grid=(S // BM, 3 * H),
        in_specs=[
            pl.BlockSpec((BM, D), lambda i, c: (i, 0)),
            pl.BlockSpec((DH, D), lambda i, c: (c, 0)),
            pl.BlockSpec((1, 1, DH), lambda i, c: (c, 0, 0)),
        ],
        out_specs=pl.BlockSpec((1, BM, DH), lambda i, c: (c, i, 0)),
        out_shape=jax.ShapeDtypeStruct((3 * H, S, DH), jnp.float32),
    )(x, Wqkv.astype(jnp.bfloat16), bqkv.reshape(3 * H, 1, DH))


def _attn_kernel(q_ref, k_ref, v_ref, o_ref):
    q = q_ref[0].astype(jnp.bfloat16)
    k = k_ref[0].astype(jnp.bfloat16)
    s = lax.dot_general(q, k, (((1,), (1,)), ((), ())),
                        preferred_element_type=jnp.float32)
    s = s * (1.0 / (DH ** 0.5))
    m = jnp.max(s, axis=-1, keepdims=True)
    p = jnp.exp(s - m)
    denom = jnp.sum(p, axis=-1, keepdims=True)
    o = lax.dot_general(p.astype(jnp.bfloat16), v_ref[0].astype(jnp.bfloat16),
                        (((1,), (0,)), ((), ())),
                        preferred_element_type=jnp.float32)
    o_ref[...] = (o / denom)[None]


def _attention(qkv):
    BM = 512
    return pl.pallas_call(
        _attn_kernel,
        grid=(H, S // BM),
        in_specs=[
            pl.BlockSpec((1, BM, DH), lambda h, i: (h, i, 0)),
            pl.BlockSpec((1, S, DH), lambda h, i: (H + h, 0, 0)),
            pl.BlockSpec((1, S, DH), lambda h, i: (2 * H + h, 0, 0)),
        ],
        out_specs=pl.BlockSpec((1, BM, DH), lambda h, i: (h, i, 0)),
        out_shape=jax.ShapeDtypeStruct((H, S, DH), jnp.float32),
    )(qkv, qkv, qkv)


def _oproj_ln_kernel(o_ref, w_ref, b_ref, r_ref, g_ref, bb_ref, out_ref):
    y = b_ref[...] + r_ref[...]
    for h in range(H):
        y = y + lax.dot_general(
            o_ref[h].astype(jnp.bfloat16), w_ref[:, h, :],
            (((1,), (1,)), ((), ())),
            preferred_element_type=jnp.float32)
    out_ref[...] = _ln_rows(y, g_ref[...], bb_ref[...])


def _oproj_ln(o, Wo, bo, resid, g, b):
    BM = 512
    return pl.pallas_call(
        _oproj_ln_kernel,
        grid=(S // BM,),
        in_specs=[
            pl.BlockSpec((H, BM, DH), lambda i: (0, i, 0)),
            pl.BlockSpec((D, H, DH), lambda i: (0, 0, 0)),
            pl.BlockSpec((1, D), lambda i: (0, 0)),
            pl.BlockSpec((BM, D), lambda i: (i, 0)),
            pl.BlockSpec((1, D), lambda i: (0, 0)),
            pl.BlockSpec((1, D), lambda i: (0, 0)),
        ],
        out_specs=pl.BlockSpec((BM, D), lambda i: (i, 0)),
        out_shape=jax.ShapeDtypeStruct((S, D), jnp.float32),
    )(o, Wo.reshape(D, H, DH).astype(jnp.bfloat16), bo.reshape(1, D), resid,
      g.reshape(1, D), b.reshape(1, D))


def _gate_kernel(x_ref, gw_ref, gb_ref, ti_ref, tp_ref):
    gs = lax.dot_general(x_ref[...], gw_ref[...], (((1,), (1,)), ((), ())),
                         preferred_element_type=jnp.float32) + gb_ref[...]
    n = gs.shape[0]
    ii = lax.broadcasted_iota(jnp.int32, (n, E), 1)
    a1 = jnp.argmax(gs, axis=-1).astype(jnp.int32)[:, None]
    m1 = jnp.max(gs, axis=-1, keepdims=True)
    gs2 = jnp.where(ii == a1, -jnp.inf, gs)
    a2 = jnp.argmax(gs2, axis=-1).astype(jnp.int32)[:, None]
    m2 = jnp.max(gs2, axis=-1, keepdims=True)
    p1 = 1.0 / (1.0 + jnp.exp(m2 - m1))
    p2 = 1.0 - p1
    ti_ref[...] = jnp.concatenate([a1, a2], axis=1)
    tp_ref[...] = jnp.concatenate([p1, p2], axis=1)


def _gate(x, gW, gb):
    BM = 1024
    return pl.pallas_call(
        _gate_kernel,
        grid=(S // BM,),
        in_specs=[
            pl.BlockSpec((BM, D), lambda i: (i, 0)),
            pl.BlockSpec((E, D), lambda i: (0, 0)),
            pl.BlockSpec((1, E), lambda i: (0, 0)),
        ],
        out_specs=[
            pl.BlockSpec((BM, TOPK), lambda i: (i, 0)),
            pl.BlockSpec((BM, TOPK), lambda i: (i, 0)),
        ],
        out_shape=[
            jax.ShapeDtypeStruct((S, TOPK), jnp.int32),
            jax.ShapeDtypeStruct((S, TOPK), jnp.float32),
        ],
    )(x, gW, gb.reshape(1, E))


def _route_body(ti_hbm, tp_hbm, disp_hbm, pslot_hbm, blk_hbm, parts_hbm,
                ti_v, tp_v, loc_d, loc_p, p0_loc, p1_loc,
                blk_v, zeros_d, zeros_p):
    cid = lax.axis_index("c")
    sid = lax.axis_index("s")
    lanes = lax.iota(jnp.int32, 16)

    @pl.when(cid == 0)
    def _():
        e = sid

        @pl.when(sid < E)
        def _():
            pltpu.sync_copy(ti_hbm, ti_v)
            pltpu.sync_copy(tp_hbm, tp_v)

            def cbody(c, cnts):
                ids = plsc.load_gather(ti_v, [c * 16 + lanes])
                return tuple(
                    cnts[j] + jnp.sum((ids == j).astype(jnp.int32))
                    for j in range(E))
            counts = lax.fori_loop(0, _NA // 16, cbody,
                                   tuple(jnp.int32(0) for _ in range(E)))
            counts = list(counts)
            pads = [((c + (_BMOE - 1)) // _BMOE) * _BMOE for c in counts]
            bases = [jnp.int32(0)]
            for j in range(E):
                bases.append(bases[j] + pads[j])
            base_e = jnp.int32(0)
            cnt_e = jnp.int32(0)
            pad_e = jnp.int32(0)
            for j in range(E):
                base_e = base_e + jnp.where(j < e, pads[j], 0)
                cnt_e = cnt_e + jnp.where(j == e, counts[j], 0)
                pad_e = pad_e + jnp.where(j == e, pads[j], 0)

            def zb(c, carry):
                idx = c * 16 + lanes
                z = jnp.zeros(16, jnp.int32)
                plsc.store_scatter(loc_d, [idx], z)
                plsc.store_scatter(loc_p, [idx], jnp.zeros(16, jnp.float32))
                plsc.store_scatter(p0_loc, [idx], z)
                plsc.store_scatter(p1_loc, [idx], z)
                return carry
            lax.fori_loop(0, _CAP // 16, zb, jnp.int32(0))

            def fbody(c, off):
                idx = c * 16 + lanes
                ids = plsc.load_gather(ti_v, [idx])
                pv = plsc.load_gather(tp_v, [idx])
                m = ids == e
                mi = m.astype(jnp.int32)
                ranks = plsc.cumsum(mi) - mi
                tok = lax.shift_right_logical(idx, 1)
                par = lax.bitwise_and(idx, 1)
                slots = base_e + off + ranks
                plsc.store_scatter(loc_d, [off + ranks], tok, mask=m)
                plsc.store_scatter(loc_p, [off + ranks], pv, mask=m)
                plsc.store_scatter(p0_loc, [tok], slots, mask=m & (par == 0))
                plsc.store_scatter(p1_loc, [tok], slots, mask=m & (par == 1))
                return off + jnp.sum(mi)
            lax.fori_loop(0, _NA // 16, fbody, jnp.int32(0))

            def wb(b, carry):
                dst = pl.multiple_of(base_e + b * _BMOE, _BMOE)
                pltpu.sync_copy(loc_d.at[pl.ds(b * _BMOE, _BMOE)],
                                disp_hbm.at[pl.ds(dst, _BMOE)])
                pltpu.sync_copy(loc_p.at[pl.ds(b * _BMOE, _BMOE)],
                                pslot_hbm.at[pl.ds(dst, _BMOE)])
                return carry
            lax.fori_loop(0, pad_e // _BMOE, wb, jnp.int32(0))
            dst0 = pl.multiple_of(e * S, S)
            dst1 = pl.multiple_of((E + e) * S, S)
            pltpu.sync_copy(p0_loc, parts_hbm.at[pl.ds(dst0, S)])
            pltpu.sync_copy(p1_loc, parts_hbm.at[pl.ds(dst1, S)])

            @pl.when(e == 0)
            def _():
                def zz(c, carry):
                    idx = c * 16 + lanes
                    plsc.store_scatter(zeros_d, [idx], jnp.zeros(16, jnp.int32))
                    plsc.store_scatter(zeros_p, [idx],
                                       jnp.zeros(16, jnp.float32))
                    return carry
                lax.fori_loop(0, _BMOE // 16, zz, jnp.int32(0))

                def tz(b, carry):
                    dst = pl.multiple_of(bases[E] + b * _BMOE, _BMOE)
                    pltpu.sync_copy(zeros_d, disp_hbm.at[pl.ds(dst, _BMOE)])
                    pltpu.sync_copy(zeros_p, pslot_hbm.at[pl.ds(dst, _BMOE)])
                    return carry
                lax.fori_loop(0, (_NSLOT - bases[E]) // _BMOE, tz,
                              jnp.int32(0))

                for half in range(_NBPAD // 16):
                    bidx = half * 16 + lanes
                    row0 = bidx * _BMOE
                    expv = jnp.zeros(16, jnp.int32)
                    for j in range(E):
                        inb = (row0 >= bases[j]) & (row0 < bases[j + 1])
                        expv = jnp.where(inb, j, expv)
                    expv = jnp.where(row0 < bases[E], expv, E - 1)
                    plsc.store_scatter(blk_v, [bidx], expv)
                pltpu.sync_copy(blk_v, blk_hbm)


def _route_sc(ti_flat, tp_flat):
    k = functools.partial(
        pl.kernel,
        mesh=plsc.VectorSubcoreMesh(core_axis_name="c", subcore_axis_name="s"),
        compiler_params=pltpu.CompilerParams(needs_layout_passes=False),
        out_type=[
            jax.ShapeDtypeStruct((_NSLOT,), jnp.int32),
            jax.ShapeDtypeStruct((_NSLOT,), jnp.float32),
            jax.ShapeDtypeStruct((_NBPAD,), jnp.int32),
            jax.ShapeDtypeStruct((2 * E * S,), jnp.int32),
        ],
        scratch_types=[
            pltpu.VMEM((_NA,), jnp.int32),
            pltpu.VMEM((_NA,), jnp.float32),
            pltpu.VMEM((_CAP,), jnp.int32),
            pltpu.VMEM((_CAP,), jnp.float32),
            pltpu.VMEM((S,), jnp.int32),
            pltpu.VMEM((S,), jnp.int32),
            pltpu.VMEM((_NBPAD,), jnp.int32),
            pltpu.VMEM((_BMOE,), jnp.int32),
            pltpu.VMEM((_BMOE,), jnp.float32),
        ],
    )(_route_body)
    return k(ti_flat, tp_flat)


def _gmm_kernel(blk_ref, xg_ref, w1_ref, b1_ref, w2_ref, b2_ref, p_ref,
                yg_ref):
    del blk_ref
    h = lax.dot_general(xg_ref[...].astype(jnp.bfloat16), w1_ref[0],
                        (((1,), (1,)), ((), ())),
                        preferred_element_type=jnp.float32) + b1_ref[0]
    h = jnp.maximum(h, 0.0).astype(jnp.bfloat16)
    y = lax.dot_general(h, w2_ref[0], (((1,), (1,)), ((), ())),
                        preferred_element_type=jnp.float32) + b2_ref[0]
    yg_ref[...] = y * p_ref[...]


def _grouped_moe(xg, W1, b1, W2, b2, pslot, blk):
    grid_spec = pltpu.PrefetchScalarGridSpec(
        num_scalar_prefetch=1,
        grid=(_NB,),
        in_specs=[
            pl.BlockSpec((_BMOE, D), lambda b, blk: (b, 0)),
            pl.BlockSpec((1, FF, D), lambda b, blk: (blk[b], 0, 0)),
            pl.BlockSpec((1, 1, FF), lambda b, blk: (blk[b], 0, 0)),
            pl.BlockSpec((1, D, FF), lambda b, blk: (blk[b], 0, 0)),
            pl.BlockSpec((1, 1, D), lambda b, blk: (blk[b], 0, 0)),
            pl.BlockSpec((_BMOE, 1), lambda b, blk: (b, 0)),
        ],
        out_specs=pl.BlockSpec((_BMOE, D), lambda b, blk: (b, 0)),
    )
    return pl.pallas_call(
        _gmm_kernel,
        grid_spec=grid_spec,
        out_shape=jax.ShapeDtypeStruct((_NSLOT, D), jnp.float32),
    )(blk, xg, W1.astype(jnp.bfloat16), b1.reshape(E, 1, FF),
      W2.astype(jnp.bfloat16), b2.reshape(E, 1, D),
      pslot.reshape(_NSLOT, 1))


def _possum_kernel(p_ref, o_ref):
    acc = p_ref[0 * S: 1 * S][None]
    for e in range(1, 2 * E):
        part = p_ref[e * S:(e + 1) * S][None]
        if e == E:
            o_ref[0:1, :] = acc
            acc = part
        else:
            acc = acc + part
    o_ref[1:2, :] = acc


def _possum(parts):
    return pl.pallas_call(
        _possum_kernel,
        grid=(1,),
        in_specs=[pl.BlockSpec((2 * E * S,), lambda i: (0,))],
        out_specs=pl.BlockSpec((2, S), lambda i: (0, 0)),
        out_shape=jax.ShapeDtypeStruct((2, S), jnp.int32),
    )(parts)


_CROWS = 8


def _combine_kernel(pos_ref, *refs):
    del pos_ref
    a = refs[:_CROWS]
    bb = refs[_CROWS:2 * _CROWS]
    x_ref, g_ref, b_ref, out_ref = refs[2 * _CROWS:]
    rows = [a[j][0, 0, :] + bb[j][0, 0, :] for j in range(_CROWS)]
    y = x_ref[...] + jnp.concatenate([r[None] for r in rows], axis=0)
    out_ref[...] = _ln_rows(y, g_ref[...], b_ref[...])


def _combine_ln(yg, pos, x, g, b):
    yg3 = yg.reshape(_NSLOT, 1, D)
    grid_spec = pltpu.PrefetchScalarGridSpec(
        num_scalar_prefetch=1,
        grid=(S // _CROWS,),
        in_specs=(
            [pl.BlockSpec((1, 1, D), functools.partial(
                lambda j, i, p: (p[0, i * _CROWS + j], 0, 0), j))
             for j in range(_CROWS)]
            + [pl.BlockSpec((1, 1, D), functools.partial(
                lambda j, i, p: (p[1, i * _CROWS + j], 0, 0), j))
               for j in range(_CROWS)]
            + [pl.BlockSpec((_CROWS, D), lambda i, p: (i, 0)),
               pl.BlockSpec((1, D), lambda i, p: (0, 0)),
               pl.BlockSpec((1, D), lambda i, p: (0, 0))]
        ),
        out_specs=pl.BlockSpec((_CROWS, D), lambda i, p: (i, 0)),
    )
    return pl.pallas_call(
        _combine_kernel,
        grid_spec=grid_spec,
        out_shape=jax.ShapeDtypeStruct((S, D), jnp.float32),
    )(pos, *([yg3] * _CROWS), *([yg3] * _CROWS), x,
      g.reshape(1, D), b.reshape(1, D))


def _head_kernel(x_ref, g_ref, b_ref, w_ref, hb_ref, o_ref):
    xb = _ln_rows(x_ref[...], g_ref[...], b_ref[...])
    o_ref[...] = lax.dot_general(
        xb.astype(jnp.bfloat16), w_ref[...], (((1,), (1,)), ((), ())),
        preferred_element_type=jnp.float32) + hb_ref[...]


def _head(x, lfg, lfb, hW, hb):
    BM, BN = 512, 1280
    return pl.pallas_call(
        _head_kernel,
        grid=(S // BM, V // BN),
        in_specs=[
            pl.BlockSpec((BM, D), lambda i, j: (i, 0)),
            pl.BlockSpec((1, D), lambda i, j: (0, 0)),
            pl.BlockSpec((1, D), lambda i, j: (0, 0)),
            pl.BlockSpec((BN, D), lambda i, j: (j, 0)),
            pl.BlockSpec((1, BN), lambda i, j: (0, j)),
        ],
        out_specs=pl.BlockSpec((BM, BN), lambda i, j: (i, j)),
        out_shape=jax.ShapeDtypeStruct((S, V), jnp.float32),
    )(x, lfg.reshape(1, D), lfb.reshape(1, D), hW.astype(jnp.bfloat16),
      hb.reshape(1, V))


def kernel(input_ids, emb, Wqkv, bqkv, Wo, bo, gW, gb, W1, b1, W2, b2,
           n1g, n1b, n2g, n2b, lfg, lfb, hW, hb):
    ids = input_ids.reshape(S).astype(jnp.int32)
    x = _sc_gather(emb, ids, S)
    for l in range(L):
        qkv = _qkv_proj(x, Wqkv[l], bqkv[l])
        o = _attention(qkv)
        x = _oproj_ln(o, Wo[l], bo[l], x, n1g[l], n1b[l])
        ti, tp = _gate(x, gW[l], gb[l])
        disp, pslot, blk, parts = _route_sc(
            ti.reshape(_NA), tp.reshape(_NA))
        pos = _possum(parts)
        xg = _sc_gather(x, disp, _NSLOT)
        yg = _grouped_moe(xg, W1[l], b1[l], W2[l], b2[l], pslot, blk)
        x = _combine_ln(yg, pos, x, n2g[l], n2b[l])
    out = _head(x, lfg, lfb, hW, hb)
    return out.reshape(B, S, V)


# speedup vs baseline: 1.1986x; 1.1986x over previous
import functools

import jax
import jax.numpy as jnp
from jax import lax
from jax.experimental import pallas as pl
from jax.experimental.pallas import tpu as pltpu
from jax.experimental.pallas import tpu_sc as plsc

V, D, H, FF, L, E, TOPK = 32000, 768, 12, 2048, 2, 8, 2
B, S = 1, 2048
DH = D // H

_EPS = 1e-5

_NA = S * TOPK
_BMOE = 512
_NB = _NA // _BMOE + E
_NBPAD = 32
_NSLOT = _NB * _BMOE
_CAP = S


def _ln_rows(y, g, b):
    m = jnp.mean(y, axis=-1, keepdims=True)
    v = jnp.mean((y - m) ** 2, axis=-1, keepdims=True)
    return (y - m) * lax.rsqrt(v + _EPS) * g + b


_GROWS = 8


def _gather_kernel(*refs):
    out_ref = refs[-1]
    for j in range(_GROWS):
        out_ref[j, :] = refs[1 + j][0, 0, :]


def _gather_rows(table, idx, n_rows):
    t3 = table.reshape(table.shape[0], 1, D)
    grid_spec = pltpu.PrefetchScalarGridSpec(
        num_scalar_prefetch=1,
        grid=(n_rows // _GROWS,),
        in_specs=[
            pl.BlockSpec((1, 1, D), functools.partial(
                lambda j, i, ids: (ids[i * _GROWS + j], 0, 0), j))
            for j in range(_GROWS)
        ],
        out_specs=pl.BlockSpec((_GROWS, D), lambda i, ids: (i, 0)),
    )
    return pl.pallas_call(
        _gather_kernel,
        grid_spec=grid_spec,
        out_shape=jax.ShapeDtypeStruct((n_rows, D), jnp.float32),
    )(idx, *([t3] * _GROWS))


_NW = 32
_GCH = 64


def _sc_gather(table, idx, n_rows):
    per = n_rows // _NW

    def body(table_hbm, idx_hbm, out_hbm, idx_v, rows_v, sem):
        wid = lax.axis_index("s") * 2 + lax.axis_index("c")
        base = wid * per
        for t in range(per // _GCH):
            off = pl.multiple_of(base + t * _GCH, _GCH)
            pltpu.sync_copy(idx_hbm.at[pl.ds(off, _GCH)], idx_v)
            pltpu.async_copy(table_hbm.at[idx_v], rows_v, sem).wait()
            pltpu.sync_copy(rows_v, out_hbm.at[pl.ds(off, _GCH)])

    k = functools.partial(
        pl.kernel,
        mesh=plsc.VectorSubcoreMesh(core_axis_name="c", subcore_axis_name="s"),
        out_type=jax.ShapeDtypeStruct((n_rows, D), jnp.float32),
        scratch_types=[
            pltpu.VMEM((_GCH,), jnp.int32),
            pltpu.VMEM((_GCH, D), jnp.float32),
            pltpu.SemaphoreType.DMA,
        ],
    )(body)
    return k(table, idx)


def _qkv_kernel(x_ref, w_ref, b_ref, o_ref):
    o_ref[...] = lax.dot_general(
        x_ref[...].astype(jnp.bfloat16), w_ref[...],
        (((1,), (1,)), ((), ())),
        preferred_element_type=jnp.float32) + b_ref[...]


def _qkv_proj(x, Wqkv, bqkv):
    BM, BN = 512, 768
    return pl.pallas_call(
        _qkv_kernel,
        grid=(S // BM, (3 * D) // BN),
        in_specs=[
            pl.BlockSpec((BM, D), lambda i, c: (i, 0)),
            pl.BlockSpec((BN, D), lambda i, c: (c, 0)),
            pl.BlockSpec((1, BN), lambda i, c: (0, c)),
        ],
        out_specs=pl.BlockSpec((BM, BN), lambda i, c: (i, c)),
        out_shape=jax.ShapeDtypeStruct((S, 3 * D), jnp.float32),
    )(x, Wqkv.astype(jnp.bfloat16), bqkv.reshape(1, 3 * D))


def _attn_kernel(q_ref, k_ref, v_ref, o_ref):
    for h in range(H):
        q = q_ref[:, h * DH:(h + 1) * DH].astype(jnp.bfloat16)
        k = k_ref[:, h * DH:(h + 1) * DH].astype(jnp.bfloat16)
        v = v_ref[:, h * DH:(h + 1) * DH].astype(jnp.bfloat16)
        s = lax.dot_general(q, k, (((1,), (1,)), ((), ())),
                            preferred_element_type=jnp.float32)
        s = s * (1.0 / (DH ** 0.5))
        m = jnp.max(s, axis=-1, keepdims=True)
        p = jnp.exp(s - m)
        denom = jnp.sum(p, axis=-1, keepdims=True)
        o = lax.dot_general(p.astype(jnp.bfloat16), v,
                            (((1,), (0,)), ((), ())),
                            preferred_element_type=jnp.float32)
        o_ref[:, h * DH:(h + 1) * DH] = o / denom


def _attention(qkv):
    BM = 512
    return pl.pallas_call(
        _attn_kernel,
        grid=(S // BM,),
        in_specs=[
            pl.BlockSpec((BM, D), lambda i: (i, 0)),
            pl.BlockSpec((S, D), lambda i: (0, 1)),
            pl.BlockSpec((S, D), lambda i: (0, 2)),
        ],
        out_specs=pl.BlockSpec((BM, D), lambda i: (i, 0)),
        out_shape=jax.ShapeDtypeStruct((S, D), jnp.float32),
    )(qkv, qkv, qkv)


def _oproj_ln_kernel(o_ref, w_ref, b_ref, r_ref, g_ref, bb_ref, out_ref):
    y = lax.dot_general(o_ref[...].astype(jnp.bfloat16), w_ref[...],
                        (((1,), (1,)), ((), ())),
                        preferred_element_type=jnp.float32)
    y = y + b_ref[...] + r_ref[...]
    out_ref[...] = _ln_rows(y, g_ref[...], bb_ref[...])


def _oproj_ln(o, Wo, bo, resid, g, b):
    BM = 512
    return pl.pallas_call(
        _oproj_ln_kernel,
        grid=(S // BM,),
        in_specs=[
            pl.BlockSpec((BM, D), lambda i: (i, 0)),
            pl.BlockSpec((D, D), lambda i: (0, 0)),
            pl.BlockSpec((1, D), lambda i: (0, 0)),
            pl.BlockSpec((BM, D), lambda i: (i, 0)),
            pl.BlockSpec((1, D), lambda i: (0, 0)),
            pl.BlockSpec((1, D), lambda i: (0, 0)),
        ],
        out_specs=pl.BlockSpec((BM, D), lambda i: (i, 0)),
        out_shape=jax.ShapeDtypeStruct((S, D), jnp.float32),
    )(o, Wo.astype(jnp.bfloat16), bo.reshape(1, D), resid,
      g.reshape(1, D), b.reshape(1, D))


def _gate_kernel(x_ref, gw_ref, gb_ref, ti_ref, tp_ref):
    gs = lax.dot_general(x_ref[...], gw_ref[...], (((1,), (1,)), ((), ())),
                         preferred_element_type=jnp.float32) + gb_ref[...]
    n = gs.shape[0]
    ii = lax.broadcasted_iota(jnp.int32, (n, E), 1)
    a1 = jnp.argmax(gs, axis=-1).astype(jnp.int32)[:, None]
    m1 = jnp.max(gs, axis=-1, keepdims=True)
    gs2 = jnp.where(ii == a1, -jnp.inf, gs)
    a2 = jnp.argmax(gs2, axis=-1).astype(jnp.int32)[:, None]
    m2 = jnp.max(gs2, axis=-1, keepdims=True)
    p1 = 1.0 / (1.0 + jnp.exp(m2 - m1))
    p2 = 1.0 - p1
    ti_ref[...] = jnp.concatenate([a1, a2], axis=1)
    tp_ref[...] = jnp.concatenate([p1, p2], axis=1)


def _gate(x, gW, gb):
    BM = 1024
    return pl.pallas_call(
        _gate_kernel,
        grid=(S // BM,),
        in_specs=[
            pl.BlockSpec((BM, D), lambda i: (i, 0)),
            pl.BlockSpec((E, D), lambda i: (0, 0)),
            pl.BlockSpec((1, E), lambda i: (0, 0)),
        ],
        out_specs=[
            pl.BlockSpec((BM, TOPK), lambda i: (i, 0)),
            pl.BlockSpec((BM, TOPK), lambda i: (i, 0)),
        ],
        out_shape=[
            jax.ShapeDtypeStruct((S, TOPK), jnp.int32),
            jax.ShapeDtypeStruct((S, TOPK), jnp.float32),
        ],
    )(x, gW, gb.reshape(1, E))


def _route_body(ti_hbm, tp_hbm, disp_hbm, pslot_hbm, blk_hbm, parts_hbm,
                ti_v, tp_v, loc_d, loc_p, p0_loc, p1_loc,
                blk_v, zeros_d, zeros_p):
    cid = lax.axis_index("c")
    sid = lax.axis_index("s")
    lanes = lax.iota(jnp.int32, 16)

    @pl.when(cid == 0)
    def _():
        e = sid

        @pl.when(sid < E)
        def _():
            pltpu.sync_copy(ti_hbm, ti_v)
            pltpu.sync_copy(tp_hbm, tp_v)

            def cbody(c, cnts):
                ids = plsc.load_gather(ti_v, [c * 16 + lanes])
                return tuple(
                    cnts[j] + plsc.all_reduce_population_count(ids == j)
                    for j in range(E))
            counts = lax.fori_loop(0, _NA // 16, cbody,
                                   tuple(jnp.zeros(16, jnp.int32)
                                         for _ in range(E)))
            counts = list(counts)
            pads = [((c + (_BMOE - 1)) // _BMOE) * _BMOE for c in counts]
            bases = [jnp.zeros(16, jnp.int32)]
            for j in range(E):
                bases.append(bases[j] + pads[j])
            base_v = jnp.zeros(16, jnp.int32)
            cnt_v = jnp.zeros(16, jnp.int32)
            pad_v = jnp.zeros(16, jnp.int32)
            for j in range(E):
                base_v = base_v + jnp.where(j < e, pads[j], 0)
                cnt_v = cnt_v + jnp.where(j == e, counts[j], 0)
                pad_v = pad_v + jnp.where(j == e, pads[j], 0)
            base_e = jnp.max(base_v)
            pad_e = jnp.max(pad_v)
            total_s = jnp.max(bases[E])

            def zb(c, carry):
                idx = c * 16 + lanes
                z = jnp.zeros(16, jnp.int32)
                plsc.store_scatter(loc_d, [idx], z)
                plsc.store_scatter(loc_p, [idx], jnp.zeros(16, jnp.float32))
                plsc.store_scatter(p0_loc, [idx], z)
                plsc.store_scatter(p1_loc, [idx], z)
                return carry
            lax.fori_loop(0, _CAP // 16, zb, jnp.int32(0))

            def fbody(c, off):
                idx = c * 16 + lanes
                ids = plsc.load_gather(ti_v, [idx])
                pv = plsc.load_gather(tp_v, [idx])
                m = ids == e
                mi = m.astype(jnp.int32)
                ranks = plsc.cumsum(mi) - mi
                tok = lax.shift_right_logical(idx, 1)
                par = lax.bitwise_and(idx, 1)
                slots = base_v + off + ranks
                plsc.store_scatter(loc_d, [off + ranks], tok, mask=m)
                plsc.store_scatter(loc_p, [off + ranks], pv, mask=m)
                plsc.store_scatter(p0_loc, [tok], slots, mask=m & (par == 0))
                plsc.store_scatter(p1_loc, [tok], slots, mask=m & (par == 1))
                return off + plsc.all_reduce_population_count(m)
            lax.fori_loop(0, _NA // 16, fbody, jnp.zeros(16, jnp.int32))

            def wb(b, carry):
                dst = pl.multiple_of(base_e + b * _BMOE, _BMOE)
                pltpu.sync_copy(loc_d.at[pl.ds(b * _BMOE, _BMOE)],
                                disp_hbm.at[pl.ds(dst, _BMOE)])
                pltpu.sync_copy(loc_p.at[pl.ds(b * _BMOE, _BMOE)],
                                pslot_hbm.at[pl.ds(dst, _BMOE)])
                return carry
            lax.fori_loop(0, pad_e // _BMOE, wb, jnp.int32(0))
            dst0 = pl.multiple_of(e * S, S)
            dst1 = pl.multiple_of((E + e) * S, S)
            pltpu.sync_copy(p0_loc, parts_hbm.at[pl.ds(dst0, S)])
            pltpu.sync_copy(p1_loc, parts_hbm.at[pl.ds(dst1, S)])

            @pl.when(e == 0)
            def _():
                def zz(c, carry):
                    idx = c * 16 + lanes
                    plsc.store_scatter(zeros_d, [idx], jnp.zeros(16, jnp.int32))
                    plsc.store_scatter(zeros_p, [idx],
                                       jnp.zeros(16, jnp.float32))
                    return carry
                lax.fori_loop(0, _BMOE // 16, zz, jnp.int32(0))

                def tz(b, carry):
                    dst = pl.multiple_of(total_s + b * _BMOE, _BMOE)
                    pltpu.sync_copy(zeros_d, disp_hbm.at[pl.ds(dst, _BMOE)])
                    pltpu.sync_copy(zeros_p, pslot_hbm.at[pl.ds(dst, _BMOE)])
                    return carry
                lax.fori_loop(0, (_NSLOT - total_s) // _BMOE, tz,
                              jnp.int32(0))

                for half in range(_NBPAD // 16):
                    bidx = half * 16 + lanes
                    row0 = bidx * _BMOE
                    expv = jnp.zeros(16, jnp.int32)
                    for j in range(E):
                        inb = (row0 >= bases[j]) & (row0 < bases[j + 1])
                        expv = jnp.where(inb, j, expv)
                    expv = jnp.where(row0 < total_s, expv, E - 1)
                    plsc.store_scatter(blk_v, [bidx], expv)
                pltpu.sync_copy(blk_v, blk_hbm)


def _route_sc(ti_flat, tp_flat):
    k = functools.partial(
        pl.kernel,
        mesh=plsc.VectorSubcoreMesh(core_axis_name="c", subcore_axis_name="s"),
        compiler_params=pltpu.CompilerParams(needs_layout_passes=False),
        out_type=[
            jax.ShapeDtypeStruct((_NSLOT,), jnp.int32),
            jax.ShapeDtypeStruct((_NSLOT,), jnp.float32),
            jax.ShapeDtypeStruct((_NBPAD,), jnp.int32),
            jax.ShapeDtypeStruct((2 * E * S,), jnp.int32),
        ],
        scratch_types=[
            pltpu.VMEM((_NA,), jnp.int32),
            pltpu.VMEM((_NA,), jnp.float32),
            pltpu.VMEM((_CAP,), jnp.int32),
            pltpu.VMEM((_CAP,), jnp.float32),
            pltpu.VMEM((S,), jnp.int32),
            pltpu.VMEM((S,), jnp.int32),
            pltpu.VMEM((_NBPAD,), jnp.int32),
            pltpu.VMEM((_BMOE,), jnp.int32),
            pltpu.VMEM((_BMOE,), jnp.float32),
        ],
    )(_route_body)
    return k(ti_flat, tp_flat)


def _gmm_kernel(blk_ref, xg_ref, w1_ref, b1_ref, w2_ref, b2_ref, p_ref,
                yg_ref):
    del blk_ref
    h = lax.dot_general(xg_ref[...].astype(jnp.bfloat16), w1_ref[0],
                        (((1,), (1,)), ((), ())),
                        preferred_element_type=jnp.float32) + b1_ref[0]
    h = jnp.maximum(h, 0.0).astype(jnp.bfloat16)
    y = lax.dot_general(h, w2_ref[0], (((1,), (1,)), ((), ())),
                        preferred_element_type=jnp.float32) + b2_ref[0]
    yg_ref[...] = y * p_ref[...]


def _grouped_moe(xg, W1, b1, W2, b2, pslot, blk):
    grid_spec = pltpu.PrefetchScalarGridSpec(
        num_scalar_prefetch=1,
        grid=(_NB,),
        in_specs=[
            pl.BlockSpec((_BMOE, D), lambda b, blk: (b, 0)),
            pl.BlockSpec((1, FF, D), lambda b, blk: (blk[b], 0, 0)),
            pl.BlockSpec((1, 1, FF), lambda b, blk: (blk[b], 0, 0)),
            pl.BlockSpec((1, D, FF), lambda b, blk: (blk[b], 0, 0)),
            pl.BlockSpec((1, 1, D), lambda b, blk: (blk[b], 0, 0)),
            pl.BlockSpec((_BMOE, 1), lambda b, blk: (b, 0)),
        ],
        out_specs=pl.BlockSpec((_BMOE, D), lambda b, blk: (b, 0)),
    )
    return pl.pallas_call(
        _gmm_kernel,
        grid_spec=grid_spec,
        out_shape=jax.ShapeDtypeStruct((_NSLOT, D), jnp.float32),
    )(blk, xg, W1.astype(jnp.bfloat16), b1.reshape(E, 1, FF),
      W2.astype(jnp.bfloat16), b2.reshape(E, 1, D),
      pslot.reshape(_NSLOT, 1))


def _possum_kernel(p_ref, o_ref):
    acc = p_ref[0 * S: 1 * S][None]
    for e in range(1, 2 * E):
        part = p_ref[e * S:(e + 1) * S][None]
        if e == E:
            o_ref[0:1, :] = acc
            acc = part
        else:
            acc = acc + part
    o_ref[1:2, :] = acc


def _possum(parts):
    return pl.pallas_call(
        _possum_kernel,
        grid=(1,),
        in_specs=[pl.BlockSpec((2 * E * S,), lambda i: (0,))],
        out_specs=pl.BlockSpec((2, S), lambda i: (0, 0)),
        out_shape=jax.ShapeDtypeStruct((2, S), jnp.int32),
    )(parts)


_CROWS = 8


def _combine_kernel(pos_ref, *refs):
    del pos_ref
    a = refs[:_CROWS]
    bb = refs[_CROWS:2 * _CROWS]
    x_ref, g_ref, b_ref, out_ref = refs[2 * _CROWS:]
    rows = [a[j][0, 0, :] + bb[j][0, 0, :] for j in range(_CROWS)]
    y = x_ref[...] + jnp.concatenate([r[None] for r in rows], axis=0)
    out_ref[...] = _ln_rows(y, g_ref[...], b_ref[...])


def _combine_ln(yg, pos, x, g, b):
    yg3 = yg.reshape(_NSLOT, 1, D)
    grid_spec = pltpu.PrefetchScalarGridSpec(
        num_scalar_prefetch=1,
        grid=(S // _CROWS,),
        in_specs=(
            [pl.BlockSpec((1, 1, D), functools.partial(
                lambda j, i, p: (p[0, i * _CROWS + j], 0, 0), j))
             for j in range(_CROWS)]
            + [pl.BlockSpec((1, 1, D), functools.partial(
                lambda j, i, p: (p[1, i * _CROWS + j], 0, 0), j))
               for j in range(_CROWS)]
            + [pl.BlockSpec((_CROWS, D), lambda i, p: (i, 0)),
               pl.BlockSpec((1, D), lambda i, p: (0, 0)),
               pl.BlockSpec((1, D), lambda i, p: (0, 0))]
        ),
        out_specs=pl.BlockSpec((_CROWS, D), lambda i, p: (i, 0)),
    )
    return pl.pallas_call(
        _combine_kernel,
        grid_spec=grid_spec,
        out_shape=jax.ShapeDtypeStruct((S, D), jnp.float32),
    )(pos, *([yg3] * _CROWS), *([yg3] * _CROWS), x,
      g.reshape(1, D), b.reshape(1, D))


def _head_kernel(x_ref, g_ref, b_ref, w_ref, hb_ref, o_ref):
    xb = _ln_rows(x_ref[...], g_ref[...], b_ref[...])
    o_ref[...] = lax.dot_general(
        xb.astype(jnp.bfloat16), w_ref[...], (((1,), (1,)), ((), ())),
        preferred_element_type=jnp.float32) + hb_ref[...]


def _head(x, lfg, lfb, hW, hb):
    BM, BN = 512, 1280
    return pl.pallas_call(
        _head_kernel,
        grid=(S // BM, V // BN),
        in_specs=[
            pl.BlockSpec((BM, D), lambda i, j: (i, 0)),
            pl.BlockSpec((1, D), lambda i, j: (0, 0)),
            pl.BlockSpec((1, D), lambda i, j: (0, 0)),
            pl.BlockSpec((BN, D), lambda i, j: (j, 0)),
            pl.BlockSpec((1, BN), lambda i, j: (0, j)),
        ],
        out_specs=pl.BlockSpec((BM, BN), lambda i, j: (i, j)),
        out_shape=jax.ShapeDtypeStruct((S, V), jnp.float32),
    )(x, lfg.reshape(1, D), lfb.reshape(1, D), hW.astype(jnp.bfloat16),
      hb.reshape(1, V))


def kernel(input_ids, emb, Wqkv, bqkv, Wo, bo, gW, gb, W1, b1, W2, b2,
           n1g, n1b, n2g, n2b, lfg, lfb, hW, hb):
    ids = input_ids.reshape(S).astype(jnp.int32)
    x = _sc_gather(emb, ids, S)
    for l in range(L):
        qkv = _qkv_proj(x, Wqkv[l], bqkv[l])
        o = _attention(qkv)
        x = _oproj_ln(o, Wo[l], bo[l], x, n1g[l], n1b[l])
        ti, tp = _gate(x, gW[l], gb[l])
        disp, pslot, blk, parts = _route_sc(
            ti.reshape(_NA), tp.reshape(_NA))
        pos = _possum(parts)
        xg = _sc_gather(x, disp, _NSLOT)
        yg = _grouped_moe(xg, W1[l], b1[l], W2[l], b2[l], pslot, blk)
        x = _combine_ln(yg, pos, x, n2g[l], n2b[l])
    out = _head(x, lfg, lfb, hW, hb)
    return out.reshape(B, S, V)


# speedup vs baseline: 1.2031x; 1.0038x over previous
import functools

import jax
import jax.numpy as jnp
from jax import lax
from jax.experimental import pallas as pl
from jax.experimental.pallas import tpu as pltpu
from jax.experimental.pallas import tpu_sc as plsc

V, D, H, FF, L, E, TOPK = 32000, 768, 12, 2048, 2, 8, 2
B, S = 1, 2048
DH = D // H

_EPS = 1e-5

_NA = S * TOPK
_BMOE = 512
_NB = _NA // _BMOE + E
_NBPAD = 32
_NSLOT = _NB * _BMOE
_CAP = S


def _ln_rows(y, g, b):
    m = jnp.mean(y, axis=-1, keepdims=True)
    v = jnp.mean((y - m) ** 2, axis=-1, keepdims=True)
    return (y - m) * lax.rsqrt(v + _EPS) * g + b


_GROWS = 8


def _gather_kernel(*refs):
    out_ref = refs[-1]
    for j in range(_GROWS):
        out_ref[j, :] = refs[1 + j][0, 0, :]


def _gather_rows(table, idx, n_rows):
    t3 = table.reshape(table.shape[0], 1, D)
    grid_spec = pltpu.PrefetchScalarGridSpec(
        num_scalar_prefetch=1,
        grid=(n_rows // _GROWS,),
        in_specs=[
            pl.BlockSpec((1, 1, D), functools.partial(
                lambda j, i, ids: (ids[i * _GROWS + j], 0, 0), j))
            for j in range(_GROWS)
        ],
        out_specs=pl.BlockSpec((_GROWS, D), lambda i, ids: (i, 0)),
    )
    return pl.pallas_call(
        _gather_kernel,
        grid_spec=grid_spec,
        out_shape=jax.ShapeDtypeStruct((n_rows, D), jnp.float32),
    )(idx, *([t3] * _GROWS))


_NW = 32
_GCH = 64


def _sc_gather(table, idx, n_rows):
    per = n_rows // _NW

    def body(table_hbm, idx_hbm, out_hbm, idx_v, rows_v, sem):
        wid = lax.axis_index("s") * 2 + lax.axis_index("c")
        base = wid * per
        for t in range(per // _GCH):
            off = pl.multiple_of(base + t * _GCH, _GCH)
            pltpu.sync_copy(idx_hbm.at[pl.ds(off, _GCH)], idx_v)
            pltpu.async_copy(table_hbm.at[idx_v], rows_v, sem).wait()
            pltpu.sync_copy(rows_v, out_hbm.at[pl.ds(off, _GCH)])

    k = functools.partial(
        pl.kernel,
        mesh=plsc.VectorSubcoreMesh(core_axis_name="c", subcore_axis_name="s"),
        out_type=jax.ShapeDtypeStruct((n_rows, D), jnp.float32),
        scratch_types=[
            pltpu.VMEM((_GCH,), jnp.int32),
            pltpu.VMEM((_GCH, D), jnp.float32),
            pltpu.SemaphoreType.DMA,
        ],
    )(body)
    return k(table, idx)


def _qkv_kernel(x_ref, w_ref, b_ref, o_ref):
    o_ref[...] = lax.dot_general(
        x_ref[...], w_ref[...],
        (((1,), (1,)), ((), ())),
        preferred_element_type=jnp.float32) + b_ref[...]


def _qkv_proj(x, Wqkv, bqkv):
    BM, BN = 512, 768
    return pl.pallas_call(
        _qkv_kernel,
        grid=(S // BM, (3 * D) // BN),
        in_specs=[
            pl.BlockSpec((BM, D), lambda i, c: (i, 0)),
            pl.BlockSpec((BN, D), lambda i, c: (c, 0)),
            pl.BlockSpec((1, BN), lambda i, c: (0, c)),
        ],
        out_specs=pl.BlockSpec((BM, BN), lambda i, c: (i, c)),
        out_shape=jax.ShapeDtypeStruct((S, 3 * D), jnp.float32),
    )(x, Wqkv, bqkv.reshape(1, 3 * D))


def _attn_kernel(q_ref, k_ref, v_ref, o_ref):
    for h in range(H):
        q = q_ref[:, h * DH:(h + 1) * DH]
        k = k_ref[:, h * DH:(h + 1) * DH]
        v = v_ref[:, h * DH:(h + 1) * DH]
        s = lax.dot_general(q, k, (((1,), (1,)), ((), ())),
                            preferred_element_type=jnp.float32)
        s = s * (1.0 / (DH ** 0.5))
        m = jnp.max(s, axis=-1, keepdims=True)
        p = jnp.exp(s - m)
        denom = jnp.sum(p, axis=-1, keepdims=True)
        o = lax.dot_general(p, v,
                            (((1,), (0,)), ((), ())),
                            preferred_element_type=jnp.float32)
        o_ref[:, h * DH:(h + 1) * DH] = o / denom


def _attention(qkv):
    BM = 512
    return pl.pallas_call(
        _attn_kernel,
        grid=(S // BM,),
        in_specs=[
            pl.BlockSpec((BM, D), lambda i: (i, 0)),
            pl.BlockSpec((S, D), lambda i: (0, 1)),
            pl.BlockSpec((S, D), lambda i: (0, 2)),
        ],
        out_specs=pl.BlockSpec((BM, D), lambda i: (i, 0)),
        out_shape=jax.ShapeDtypeStruct((S, D), jnp.float32),
    )(qkv, qkv, qkv)


def _oproj_ln_kernel(o_ref, w_ref, b_ref, r_ref, g_ref, bb_ref, out_ref):
    y = lax.dot_general(o_ref[...], w_ref[...],
                        (((1,), (1,)), ((), ())),
                        preferred_element_type=jnp.float32)
    y = y + b_ref[...] + r_ref[...]
    out_ref[...] = _ln_rows(y, g_ref[...], bb_ref[...])


def _oproj_ln(o, Wo, bo, resid, g, b):
    BM = 512
    return pl.pallas_call(
        _oproj_ln_kernel,
        grid=(S // BM,),
        in_specs=[
            pl.BlockSpec((BM, D), lambda i: (i, 0)),
            pl.BlockSpec((D, D), lambda i: (0, 0)),
            pl.BlockSpec((1, D), lambda i: (0, 0)),
            pl.BlockSpec((BM, D), lambda i: (i, 0)),
            pl.BlockSpec((1, D), lambda i: (0, 0)),
            pl.BlockSpec((1, D), lambda i: (0, 0)),
        ],
        out_specs=pl.BlockSpec((BM, D), lambda i: (i, 0)),
        out_shape=jax.ShapeDtypeStruct((S, D), jnp.float32),
    )(o, Wo, bo.reshape(1, D), resid,
      g.reshape(1, D), b.reshape(1, D))


def _gate_kernel(x_ref, gw_ref, gb_ref, ti_ref, tp_ref):
    gs = lax.dot_general(x_ref[...], gw_ref[...], (((1,), (1,)), ((), ())),
                         preferred_element_type=jnp.float32) + gb_ref[...]
    n = gs.shape[0]
    ii = lax.broadcasted_iota(jnp.int32, (n, E), 1)
    a1 = jnp.argmax(gs, axis=-1).astype(jnp.int32)[:, None]
    m1 = jnp.max(gs, axis=-1, keepdims=True)
    gs2 = jnp.where(ii == a1, -jnp.inf, gs)
    a2 = jnp.argmax(gs2, axis=-1).astype(jnp.int32)[:, None]
    m2 = jnp.max(gs2, axis=-1, keepdims=True)
    p1 = 1.0 / (1.0 + jnp.exp(m2 - m1))
    p2 = 1.0 - p1
    ti_ref[...] = jnp.concatenate([a1, a2], axis=1)
    tp_ref[...] = jnp.concatenate([p1, p2], axis=1)


def _gate(x, gW, gb):
    BM = 1024
    return pl.pallas_call(
        _gate_kernel,
        grid=(S // BM,),
        in_specs=[
            pl.BlockSpec((BM, D), lambda i: (i, 0)),
            pl.BlockSpec((E, D), lambda i: (0, 0)),
            pl.BlockSpec((1, E), lambda i: (0, 0)),
        ],
        out_specs=[
            pl.BlockSpec((BM, TOPK), lambda i: (i, 0)),
            pl.BlockSpec((BM, TOPK), lambda i: (i, 0)),
        ],
        out_shape=[
            jax.ShapeDtypeStruct((S, TOPK), jnp.int32),
            jax.ShapeDtypeStruct((S, TOPK), jnp.float32),
        ],
    )(x, gW, gb.reshape(1, E))


def _route_body(ti_hbm, tp_hbm, disp_hbm, pslot_hbm, blk_hbm, parts_hbm,
                ti_v, tp_v, loc_d, loc_p, p0_loc, p1_loc,
                blk_v, zeros_d, zeros_p):
    cid = lax.axis_index("c")
    sid = lax.axis_index("s")
    lanes = lax.iota(jnp.int32, 16)

    @pl.when(cid == 0)
    def _():
        e = sid

        @pl.when(sid < E)
        def _():
            pltpu.sync_copy(ti_hbm, ti_v)
            pltpu.sync_copy(tp_hbm, tp_v)

            def cbody(c, cnts):
                cnts = list(cnts)
                for u in range(16):
                    ids = plsc.load_gather(ti_v, [(c * 16 + u) * 16 + lanes])
                    for j in range(E):
                        cnts[j] = cnts[j] + plsc.all_reduce_population_count(
                            ids == j)
                return tuple(cnts)
            counts = lax.fori_loop(0, _NA // 256, cbody,
                                   tuple(jnp.zeros(16, jnp.int32)
                                         for _ in range(E)))
            counts = list(counts)
            pads = [((c + (_BMOE - 1)) // _BMOE) * _BMOE for c in counts]
            bases = [jnp.zeros(16, jnp.int32)]
            for j in range(E):
                bases.append(bases[j] + pads[j])
            base_v = jnp.zeros(16, jnp.int32)
            cnt_v = jnp.zeros(16, jnp.int32)
            pad_v = jnp.zeros(16, jnp.int32)
            for j in range(E):
                base_v = base_v + jnp.where(j < e, pads[j], 0)
                cnt_v = cnt_v + jnp.where(j == e, counts[j], 0)
                pad_v = pad_v + jnp.where(j == e, pads[j], 0)
            base_e = jnp.max(base_v)
            pad_e = jnp.max(pad_v)
            total_s = jnp.max(bases[E])

            def zb(c, carry):
                z = jnp.zeros(16, jnp.int32)
                zf = jnp.zeros(16, jnp.float32)
                for u in range(16):
                    idx = (c * 16 + u) * 16 + lanes
                    plsc.store_scatter(loc_d, [idx], z)
                    plsc.store_scatter(loc_p, [idx], zf)
                    plsc.store_scatter(p0_loc, [idx], z)
                    plsc.store_scatter(p1_loc, [idx], z)
                return carry
            lax.fori_loop(0, _CAP // 256, zb, jnp.int32(0))

            def fbody(c, off):
                for u in range(16):
                    idx = (c * 16 + u) * 16 + lanes
                    ids = plsc.load_gather(ti_v, [idx])
                    pv = plsc.load_gather(tp_v, [idx])
                    m = ids == e
                    mi = m.astype(jnp.int32)
                    ranks = plsc.cumsum(mi) - mi
                    tok = lax.shift_right_logical(idx, 1)
                    par = lax.bitwise_and(idx, 1)
                    slots = base_v + off + ranks
                    plsc.store_scatter(loc_d, [off + ranks], tok, mask=m)
                    plsc.store_scatter(loc_p, [off + ranks], pv, mask=m)
                    plsc.store_scatter(p0_loc, [tok], slots,
                                       mask=m & (par == 0))
                    plsc.store_scatter(p1_loc, [tok], slots,
                                       mask=m & (par == 1))
                    off = off + plsc.all_reduce_population_count(m)
                return off
            lax.fori_loop(0, _NA // 256, fbody, jnp.zeros(16, jnp.int32))

            def wb(b, carry):
                dst = pl.multiple_of(base_e + b * _BMOE, _BMOE)
                pltpu.sync_copy(loc_d.at[pl.ds(b * _BMOE, _BMOE)],
                                disp_hbm.at[pl.ds(dst, _BMOE)])
                pltpu.sync_copy(loc_p.at[pl.ds(b * _BMOE, _BMOE)],
                                pslot_hbm.at[pl.ds(dst, _BMOE)])
                return carry
            lax.fori_loop(0, pad_e // _BMOE, wb, jnp.int32(0))
            dst0 = pl.multiple_of(e * S, S)
            dst1 = pl.multiple_of((E + e) * S, S)
            pltpu.sync_copy(p0_loc, parts_hbm.at[pl.ds(dst0, S)])
            pltpu.sync_copy(p1_loc, parts_hbm.at[pl.ds(dst1, S)])

            @pl.when(e == 0)
            def _():
                def zz(c, carry):
                    idx = c * 16 + lanes
                    plsc.store_scatter(zeros_d, [idx], jnp.zeros(16, jnp.int32))
                    plsc.store_scatter(zeros_p, [idx],
                                       jnp.zeros(16, jnp.float32))
                    return carry
                lax.fori_loop(0, _BMOE // 16, zz, jnp.int32(0))

                def tz(b, carry):
                    dst = pl.multiple_of(total_s + b * _BMOE, _BMOE)
                    pltpu.sync_copy(zeros_d, disp_hbm.at[pl.ds(dst, _BMOE)])
                    pltpu.sync_copy(zeros_p, pslot_hbm.at[pl.ds(dst, _BMOE)])
                    return carry
                lax.fori_loop(0, (_NSLOT - total_s) // _BMOE, tz,
                              jnp.int32(0))

                for half in range(_NBPAD // 16):
                    bidx = half * 16 + lanes
                    row0 = bidx * _BMOE
                    expv = jnp.zeros(16, jnp.int32)
                    for j in range(E):
                        inb = (row0 >= bases[j]) & (row0 < bases[j + 1])
                        expv = jnp.where(inb, j, expv)
                    expv = jnp.where(row0 < total_s, expv, E - 1)
                    plsc.store_scatter(blk_v, [bidx], expv)
                pltpu.sync_copy(blk_v, blk_hbm)


def _route_sc(ti_flat, tp_flat):
    k = functools.partial(
        pl.kernel,
        mesh=plsc.VectorSubcoreMesh(core_axis_name="c", subcore_axis_name="s"),
        compiler_params=pltpu.CompilerParams(needs_layout_passes=False),
        out_type=[
            jax.ShapeDtypeStruct((_NSLOT,), jnp.int32),
            jax.ShapeDtypeStruct((_NSLOT,), jnp.float32),
            jax.ShapeDtypeStruct((_NBPAD,), jnp.int32),
            jax.ShapeDtypeStruct((2 * E * S,), jnp.int32),
        ],
        scratch_types=[
            pltpu.VMEM((_NA,), jnp.int32),
            pltpu.VMEM((_NA,), jnp.float32),
            pltpu.VMEM((_CAP,), jnp.int32),
            pltpu.VMEM((_CAP,), jnp.float32),
            pltpu.VMEM((S,), jnp.int32),
            pltpu.VMEM((S,), jnp.int32),
            pltpu.VMEM((_NBPAD,), jnp.int32),
            pltpu.VMEM((_BMOE,), jnp.int32),
            pltpu.VMEM((_BMOE,), jnp.float32),
        ],
    )(_route_body)
    return k(ti_flat, tp_flat)


def _gmm_kernel(blk_ref, xg_ref, w1_ref, b1_ref, w2_ref, b2_ref, p_ref,
                yg_ref):
    del blk_ref
    h = lax.dot_general(xg_ref[...], w1_ref[0],
                        (((1,), (1,)), ((), ())),
                        preferred_element_type=jnp.float32) + b1_ref[0]
    h = jnp.maximum(h, 0.0)
    y = lax.dot_general(h, w2_ref[0], (((1,), (1,)), ((), ())),
                        preferred_element_type=jnp.float32) + b2_ref[0]
    yg_ref[...] = y * p_ref[...]


def _grouped_moe(xg, W1, b1, W2, b2, pslot, blk):
    grid_spec = pltpu.PrefetchScalarGridSpec(
        num_scalar_prefetch=1,
        grid=(_NB,),
        in_specs=[
            pl.BlockSpec((_BMOE, D), lambda b, blk: (b, 0)),
            pl.BlockSpec((1, FF, D), lambda b, blk: (blk[b], 0, 0)),
            pl.BlockSpec((1, 1, FF), lambda b, blk: (blk[b], 0, 0)),
            pl.BlockSpec((1, D, FF), lambda b, blk: (blk[b], 0, 0)),
            pl.BlockSpec((1, 1, D), lambda b, blk: (blk[b], 0, 0)),
            pl.BlockSpec((_BMOE, 1), lambda b, blk: (b, 0)),
        ],
        out_specs=pl.BlockSpec((_BMOE, D), lambda b, blk: (b, 0)),
    )
    return pl.pallas_call(
        _gmm_kernel,
        grid_spec=grid_spec,
        out_shape=jax.ShapeDtypeStruct((_NSLOT, D), jnp.float32),
    )(blk, xg, W1, b1.reshape(E, 1, FF),
      W2, b2.reshape(E, 1, D),
      pslot.reshape(_NSLOT, 1))


def _possum_kernel(p_ref, o_ref):
    acc = p_ref[0 * S: 1 * S][None]
    for e in range(1, 2 * E):
        part = p_ref[e * S:(e + 1) * S][None]
        if e == E:
            o_ref[0:1, :] = acc
            acc = part
        else:
            acc = acc + part
    o_ref[1:2, :] = acc


def _possum(parts):
    return pl.pallas_call(
        _possum_kernel,
        grid=(1,),
        in_specs=[pl.BlockSpec((2 * E * S,), lambda i: (0,))],
        out_specs=pl.BlockSpec((2, S), lambda i: (0, 0)),
        out_shape=jax.ShapeDtypeStruct((2, S), jnp.int32),
    )(parts)


_CROWS = 8


def _combine_kernel(pos_ref, *refs):
    del pos_ref
    a = refs[:_CROWS]
    bb = refs[_CROWS:2 * _CROWS]
    x_ref, g_ref, b_ref, out_ref = refs[2 * _CROWS:]
    rows = [a[j][0, 0, :] + bb[j][0, 0, :] for j in range(_CROWS)]
    y = x_ref[...] + jnp.concatenate([r[None] for r in rows], axis=0)
    out_ref[...] = _ln_rows(y, g_ref[...], b_ref[...])


def _combine_ln(yg, pos, x, g, b):
    yg3 = yg.reshape(_NSLOT, 1, D)
    grid_spec = pltpu.PrefetchScalarGridSpec(
        num_scalar_prefetch=1,
        grid=(S // _CROWS,),
        in_specs=(
            [pl.BlockSpec((1, 1, D), functools.partial(
                lambda j, i, p: (p[0, i * _CROWS + j], 0, 0), j))
             for j in range(_CROWS)]
            + [pl.BlockSpec((1, 1, D), functools.partial(
                lambda j, i, p: (p[1, i * _CROWS + j], 0, 0), j))
               for j in range(_CROWS)]
            + [pl.BlockSpec((_CROWS, D), lambda i, p: (i, 0)),
               pl.BlockSpec((1, D), lambda i, p: (0, 0)),
               pl.BlockSpec((1, D), lambda i, p: (0, 0))]
        ),
        out_specs=pl.BlockSpec((_CROWS, D), lambda i, p: (i, 0)),
    )
    return pl.pallas_call(
        _combine_kernel,
        grid_spec=grid_spec,
        out_shape=jax.ShapeDtypeStruct((S, D), jnp.float32),
    )(pos, *([yg3] * _CROWS), *([yg3] * _CROWS), x,
      g.reshape(1, D), b.reshape(1, D))


def _head_kernel(x_ref, g_ref, b_ref, w_ref, hb_ref, o_ref):
    xb = _ln_rows(x_ref[...], g_ref[...], b_ref[...])
    o_ref[...] = lax.dot_general(
        xb, w_ref[...], (((1,), (1,)), ((), ())),
        preferred_element_type=jnp.float32) + hb_ref[...]


def _head(x, lfg, lfb, hW, hb):
    BM, BN = 512, 1280
    return pl.pallas_call(
        _head_kernel,
        grid=(S // BM, V // BN),
        in_specs=[
            pl.BlockSpec((BM, D), lambda i, j: (i, 0)),
            pl.BlockSpec((1, D), lambda i, j: (0, 0)),
            pl.BlockSpec((1, D), lambda i, j: (0, 0)),
            pl.BlockSpec((BN, D), lambda i, j: (j, 0)),
            pl.BlockSpec((1, BN), lambda i, j: (0, j)),
        ],
        out_specs=pl.BlockSpec((BM, BN), lambda i, j: (i, j)),
        out_shape=jax.ShapeDtypeStruct((S, V), jnp.float32),
    )(x, lfg.reshape(1, D), lfb.reshape(1, D), hW,
      hb.reshape(1, V))


def kernel(input_ids, emb, Wqkv, bqkv, Wo, bo, gW, gb, W1, b1, W2, b2,
           n1g, n1b, n2g, n2b, lfg, lfb, hW, hb):
    ids = input_ids.reshape(S).astype(jnp.int32)
    x = _sc_gather(emb, ids, S)
    for l in range(L):
        qkv = _qkv_proj(x, Wqkv[l], bqkv[l])
        o = _attention(qkv)
        x = _oproj_ln(o, Wo[l], bo[l], x, n1g[l], n1b[l])
        ti, tp = _gate(x, gW[l], gb[l])
        disp, pslot, blk, parts = _route_sc(
            ti.reshape(_NA), tp.reshape(_NA))
        pos = _possum(parts)
        xg = _sc_gather(x, disp, _NSLOT)
        yg = _grouped_moe(xg, W1[l], b1[l], W2[l], b2[l], pslot, blk)
        x = _combine_ln(yg, pos, x, n2g[l], n2b[l])
    out = _head(x, lfg, lfb, hW, hb)
    return out.reshape(B, S, V)


# speedup vs baseline: 1.2326x; 1.0245x over previous
import functools

import jax
import jax.numpy as jnp
from jax import lax
from jax.experimental import pallas as pl
from jax.experimental.pallas import tpu as pltpu
from jax.experimental.pallas import tpu_sc as plsc

V, D, H, FF, L, E, TOPK = 32000, 768, 12, 2048, 2, 8, 2
B, S = 1, 2048
DH = D // H

_EPS = 1e-5

_NA = S * TOPK
_BMOE = 512
_NB = _NA // _BMOE + E
_NBPAD = 32
_NSLOT = _NB * _BMOE
_CAP = S


def _ln_rows(y, g, b):
    m = jnp.mean(y, axis=-1, keepdims=True)
    v = jnp.mean((y - m) ** 2, axis=-1, keepdims=True)
    return (y - m) * lax.rsqrt(v + _EPS) * g + b


_GROWS = 8


def _gather_kernel(*refs):
    out_ref = refs[-1]
    for j in range(_GROWS):
        out_ref[j, :] = refs[1 + j][0, 0, :]


def _gather_rows(table, idx, n_rows):
    t3 = table.reshape(table.shape[0], 1, D)
    grid_spec = pltpu.PrefetchScalarGridSpec(
        num_scalar_prefetch=1,
        grid=(n_rows // _GROWS,),
        in_specs=[
            pl.BlockSpec((1, 1, D), functools.partial(
                lambda j, i, ids: (ids[i * _GROWS + j], 0, 0), j))
            for j in range(_GROWS)
        ],
        out_specs=pl.BlockSpec((_GROWS, D), lambda i, ids: (i, 0)),
    )
    return pl.pallas_call(
        _gather_kernel,
        grid_spec=grid_spec,
        out_shape=jax.ShapeDtypeStruct((n_rows, D), jnp.float32),
    )(idx, *([t3] * _GROWS))


_NW = 32
_GCH = 64


def _sc_gather(table, idx, n_rows):
    per = n_rows // _NW

    def body(table_hbm, idx_hbm, out_hbm, idx_v, rows_v, sem):
        wid = lax.axis_index("s") * 2 + lax.axis_index("c")
        base = wid * per
        for t in range(per // _GCH):
            off = pl.multiple_of(base + t * _GCH, _GCH)
            pltpu.sync_copy(idx_hbm.at[pl.ds(off, _GCH)], idx_v)
            pltpu.async_copy(table_hbm.at[idx_v], rows_v, sem).wait()
            pltpu.sync_copy(rows_v, out_hbm.at[pl.ds(off, _GCH)])

    k = functools.partial(
        pl.kernel,
        mesh=plsc.VectorSubcoreMesh(core_axis_name="c", subcore_axis_name="s"),
        out_type=jax.ShapeDtypeStruct((n_rows, D), jnp.float32),
        scratch_types=[
            pltpu.VMEM((_GCH,), jnp.int32),
            pltpu.VMEM((_GCH, D), jnp.float32),
            pltpu.SemaphoreType.DMA,
        ],
    )(body)
    return k(table, idx)


def _qkv_kernel(x_ref, w_ref, b_ref, o_ref):
    o_ref[...] = lax.dot_general(
        x_ref[...], w_ref[...],
        (((1,), (1,)), ((), ())),
        preferred_element_type=jnp.float32) + b_ref[...]


def _qkv_proj(x, Wqkv, bqkv):
    BM, BN = 512, 768
    return pl.pallas_call(
        _qkv_kernel,
        grid=(S // BM, (3 * D) // BN),
        in_specs=[
            pl.BlockSpec((BM, D), lambda i, c: (i, 0)),
            pl.BlockSpec((BN, D), lambda i, c: (c, 0)),
            pl.BlockSpec((1, BN), lambda i, c: (0, c)),
        ],
        out_specs=pl.BlockSpec((BM, BN), lambda i, c: (i, c)),
        out_shape=jax.ShapeDtypeStruct((S, 3 * D), jnp.float32),
    )(x, Wqkv, bqkv.reshape(1, 3 * D))


def _attn_kernel(q_ref, k_ref, v_ref, o_ref):
    for h in range(H):
        q = q_ref[:, h * DH:(h + 1) * DH]
        k = k_ref[:, h * DH:(h + 1) * DH]
        v = v_ref[:, h * DH:(h + 1) * DH]
        s = lax.dot_general(q, k, (((1,), (1,)), ((), ())),
                            preferred_element_type=jnp.float32)
        s = s * (1.0 / (DH ** 0.5))
        m = jnp.max(s, axis=-1, keepdims=True)
        p = jnp.exp(s - m)
        denom = jnp.sum(p, axis=-1, keepdims=True)
        o = lax.dot_general(p, v,
                            (((1,), (0,)), ((), ())),
                            preferred_element_type=jnp.float32)
        o_ref[:, h * DH:(h + 1) * DH] = o / denom


def _attention(qkv):
    BM = 512
    return pl.pallas_call(
        _attn_kernel,
        grid=(S // BM,),
        in_specs=[
            pl.BlockSpec((BM, D), lambda i: (i, 0)),
            pl.BlockSpec((S, D), lambda i: (0, 1)),
            pl.BlockSpec((S, D), lambda i: (0, 2)),
        ],
        out_specs=pl.BlockSpec((BM, D), lambda i: (i, 0)),
        out_shape=jax.ShapeDtypeStruct((S, D), jnp.float32),
    )(qkv, qkv, qkv)


def _oproj_ln_kernel(o_ref, w_ref, b_ref, r_ref, g_ref, bb_ref, out_ref):
    y = lax.dot_general(o_ref[...], w_ref[...],
                        (((1,), (1,)), ((), ())),
                        preferred_element_type=jnp.float32)
    y = y + b_ref[...] + r_ref[...]
    out_ref[...] = _ln_rows(y, g_ref[...], bb_ref[...])


def _oproj_ln(o, Wo, bo, resid, g, b):
    BM = 512
    return pl.pallas_call(
        _oproj_ln_kernel,
        grid=(S // BM,),
        in_specs=[
            pl.BlockSpec((BM, D), lambda i: (i, 0)),
            pl.BlockSpec((D, D), lambda i: (0, 0)),
            pl.BlockSpec((1, D), lambda i: (0, 0)),
            pl.BlockSpec((BM, D), lambda i: (i, 0)),
            pl.BlockSpec((1, D), lambda i: (0, 0)),
            pl.BlockSpec((1, D), lambda i: (0, 0)),
        ],
        out_specs=pl.BlockSpec((BM, D), lambda i: (i, 0)),
        out_shape=jax.ShapeDtypeStruct((S, D), jnp.float32),
    )(o, Wo, bo.reshape(1, D), resid,
      g.reshape(1, D), b.reshape(1, D))


def _gate_kernel(x_ref, gw_ref, gb_ref, ti_ref, tp_ref):
    gs = lax.dot_general(x_ref[...], gw_ref[...], (((1,), (1,)), ((), ())),
                         preferred_element_type=jnp.float32) + gb_ref[...]
    n = gs.shape[0]
    ii = lax.broadcasted_iota(jnp.int32, (n, E), 1)
    a1 = jnp.argmax(gs, axis=-1).astype(jnp.int32)[:, None]
    m1 = jnp.max(gs, axis=-1, keepdims=True)
    gs2 = jnp.where(ii == a1, -jnp.inf, gs)
    a2 = jnp.argmax(gs2, axis=-1).astype(jnp.int32)[:, None]
    m2 = jnp.max(gs2, axis=-1, keepdims=True)
    p1 = 1.0 / (1.0 + jnp.exp(m2 - m1))
    p2 = 1.0 - p1
    ti_ref[...] = jnp.concatenate([a1, a2], axis=1)
    tp_ref[...] = jnp.concatenate([p1, p2], axis=1)


def _gate(x, gW, gb):
    BM = 1024
    return pl.pallas_call(
        _gate_kernel,
        grid=(S // BM,),
        in_specs=[
            pl.BlockSpec((BM, D), lambda i: (i, 0)),
            pl.BlockSpec((E, D), lambda i: (0, 0)),
            pl.BlockSpec((1, E), lambda i: (0, 0)),
        ],
        out_specs=[
            pl.BlockSpec((BM, TOPK), lambda i: (i, 0)),
            pl.BlockSpec((BM, TOPK), lambda i: (i, 0)),
        ],
        out_shape=[
            jax.ShapeDtypeStruct((S, TOPK), jnp.int32),
            jax.ShapeDtypeStruct((S, TOPK), jnp.float32),
        ],
    )(x, gW, gb.reshape(1, E))


def _route_body(ti_hbm, tp_hbm, disp_hbm, pslot_hbm, blk_hbm, parts_hbm,
                ti_v, tp_v, loc_d, loc_p, p0_loc, p1_loc,
                blk_v, zeros_d, zeros_p):
    cid = lax.axis_index("c")
    sid = lax.axis_index("s")
    lanes = lax.iota(jnp.int32, 16)

    @pl.when(cid == 0)
    def _():
        e = sid

        @pl.when(sid < E)
        def _():
            pltpu.sync_copy(ti_hbm, ti_v)
            pltpu.sync_copy(tp_hbm, tp_v)

            def cbody(c, cnts):
                cnts = list(cnts)
                for u in range(16):
                    ids = plsc.load_gather(ti_v, [(c * 16 + u) * 16 + lanes])
                    for j in range(E):
                        cnts[j] = cnts[j] + plsc.all_reduce_population_count(
                            ids == j)
                return tuple(cnts)
            counts = lax.fori_loop(0, _NA // 256, cbody,
                                   tuple(jnp.zeros(16, jnp.int32)
                                         for _ in range(E)))
            counts = list(counts)
            pads = [((c + (_BMOE - 1)) // _BMOE) * _BMOE for c in counts]
            bases = [jnp.zeros(16, jnp.int32)]
            for j in range(E):
                bases.append(bases[j] + pads[j])
            base_v = jnp.zeros(16, jnp.int32)
            cnt_v = jnp.zeros(16, jnp.int32)
            pad_v = jnp.zeros(16, jnp.int32)
            for j in range(E):
                base_v = base_v + jnp.where(j < e, pads[j], 0)
                cnt_v = cnt_v + jnp.where(j == e, counts[j], 0)
                pad_v = pad_v + jnp.where(j == e, pads[j], 0)
            base_e = jnp.max(base_v)
            pad_e = jnp.max(pad_v)
            total_s = jnp.max(bases[E])

            def zb(c, carry):
                z = jnp.zeros(16, jnp.int32)
                zf = jnp.zeros(16, jnp.float32)
                for u in range(16):
                    idx = (c * 16 + u) * 16 + lanes
                    plsc.store_scatter(loc_d, [idx], z)
                    plsc.store_scatter(loc_p, [idx], zf)
                    plsc.store_scatter(p0_loc, [idx], z)
                    plsc.store_scatter(p1_loc, [idx], z)
                return carry
            lax.fori_loop(0, _CAP // 256, zb, jnp.int32(0))

            def fbody(c, off):
                for u in range(16):
                    idx = (c * 16 + u) * 16 + lanes
                    ids = plsc.load_gather(ti_v, [idx])
                    pv = plsc.load_gather(tp_v, [idx])
                    m = ids == e
                    mi = m.astype(jnp.int32)
                    ranks = plsc.cumsum(mi) - mi
                    tok = lax.shift_right_logical(idx, 1)
                    par = lax.bitwise_and(idx, 1)
                    slots = base_v + off + ranks
                    plsc.store_scatter(loc_d, [off + ranks], tok, mask=m)
                    plsc.store_scatter(loc_p, [off + ranks], pv, mask=m)
                    plsc.store_scatter(p0_loc, [tok], slots,
                                       mask=m & (par == 0))
                    plsc.store_scatter(p1_loc, [tok], slots,
                                       mask=m & (par == 1))
                    off = off + plsc.all_reduce_population_count(m)
                return off
            lax.fori_loop(0, _NA // 256, fbody, jnp.zeros(16, jnp.int32))

            def wb(b, carry):
                dst = pl.multiple_of(base_e + b * _BMOE, _BMOE)
                pltpu.sync_copy(loc_d.at[pl.ds(b * _BMOE, _BMOE)],
                                disp_hbm.at[pl.ds(dst, _BMOE)])
                pltpu.sync_copy(loc_p.at[pl.ds(b * _BMOE, _BMOE)],
                                pslot_hbm.at[pl.ds(dst, _BMOE)])
                return carry
            lax.fori_loop(0, pad_e // _BMOE, wb, jnp.int32(0))
            dst0 = pl.multiple_of(e * S, S)
            dst1 = pl.multiple_of((E + e) * S, S)
            pltpu.sync_copy(p0_loc, parts_hbm.at[pl.ds(dst0, S)])
            pltpu.sync_copy(p1_loc, parts_hbm.at[pl.ds(dst1, S)])

            @pl.when(e == 0)
            def _():
                def zz(c, carry):
                    idx = c * 16 + lanes
                    plsc.store_scatter(zeros_d, [idx], jnp.zeros(16, jnp.int32))
                    plsc.store_scatter(zeros_p, [idx],
                                       jnp.zeros(16, jnp.float32))
                    return carry
                lax.fori_loop(0, _BMOE // 16, zz, jnp.int32(0))

                def tz(b, carry):
                    dst = pl.multiple_of(total_s + b * _BMOE, _BMOE)
                    pltpu.sync_copy(zeros_d, disp_hbm.at[pl.ds(dst, _BMOE)])
                    pltpu.sync_copy(zeros_p, pslot_hbm.at[pl.ds(dst, _BMOE)])
                    return carry
                lax.fori_loop(0, (_NSLOT - total_s) // _BMOE, tz,
                              jnp.int32(0))

                for half in range(_NBPAD // 16):
                    bidx = half * 16 + lanes
                    row0 = bidx * _BMOE
                    expv = jnp.zeros(16, jnp.int32)
                    for j in range(E):
                        inb = (row0 >= bases[j]) & (row0 < bases[j + 1])
                        expv = jnp.where(inb, j, expv)
                    expv = jnp.where(row0 < total_s, expv, E - 1)
                    plsc.store_scatter(blk_v, [bidx], expv)
                pltpu.sync_copy(blk_v, blk_hbm)


def _route_sc(ti_flat, tp_flat):
    k = functools.partial(
        pl.kernel,
        mesh=plsc.VectorSubcoreMesh(core_axis_name="c", subcore_axis_name="s"),
        compiler_params=pltpu.CompilerParams(needs_layout_passes=False),
        out_type=[
            jax.ShapeDtypeStruct((_NSLOT,), jnp.int32),
            jax.ShapeDtypeStruct((_NSLOT,), jnp.float32),
            jax.ShapeDtypeStruct((_NBPAD,), jnp.int32),
            jax.ShapeDtypeStruct((2 * E * S,), jnp.int32),
        ],
        scratch_types=[
            pltpu.VMEM((_NA,), jnp.int32),
            pltpu.VMEM((_NA,), jnp.float32),
            pltpu.VMEM((_CAP,), jnp.int32),
            pltpu.VMEM((_CAP,), jnp.float32),
            pltpu.VMEM((S,), jnp.int32),
            pltpu.VMEM((S,), jnp.int32),
            pltpu.VMEM((_NBPAD,), jnp.int32),
            pltpu.VMEM((_BMOE,), jnp.int32),
            pltpu.VMEM((_BMOE,), jnp.float32),
        ],
    )(_route_body)
    return k(ti_flat, tp_flat)


def _gmm_kernel(blk_ref, xg_ref, w1_ref, b1_ref, w2_ref, b2_ref, p_ref,
                yg_ref):
    del blk_ref
    h = lax.dot_general(xg_ref[...], w1_ref[0],
                        (((1,), (1,)), ((), ())),
                        preferred_element_type=jnp.float32) + b1_ref[0]
    h = jnp.maximum(h, 0.0)
    y = lax.dot_general(h, w2_ref[0], (((1,), (1,)), ((), ())),
                        preferred_element_type=jnp.float32) + b2_ref[0]
    yg_ref[...] = (y * p_ref[...])[:, None, :]


def _grouped_moe(xg, W1, b1, W2, b2, pslot, blk):
    grid_spec = pltpu.PrefetchScalarGridSpec(
        num_scalar_prefetch=1,
        grid=(_NB,),
        in_specs=[
            pl.BlockSpec((_BMOE, D), lambda b, blk: (b, 0)),
            pl.BlockSpec((1, FF, D), lambda b, blk: (blk[b], 0, 0)),
            pl.BlockSpec((1, 1, FF), lambda b, blk: (blk[b], 0, 0)),
            pl.BlockSpec((1, D, FF), lambda b, blk: (blk[b], 0, 0)),
            pl.BlockSpec((1, 1, D), lambda b, blk: (blk[b], 0, 0)),
            pl.BlockSpec((_BMOE, 1), lambda b, blk: (b, 0)),
        ],
        out_specs=pl.BlockSpec((_BMOE, 1, D), lambda b, blk: (b, 0, 0)),
    )
    return pl.pallas_call(
        _gmm_kernel,
        grid_spec=grid_spec,
        out_shape=jax.ShapeDtypeStruct((_NSLOT, 1, D), jnp.float32),
    )(blk, xg, W1, b1.reshape(E, 1, FF),
      W2, b2.reshape(E, 1, D),
      pslot.reshape(_NSLOT, 1))


def _possum_kernel(p_ref, o_ref):
    acc = p_ref[0 * S: 1 * S][None]
    for e in range(1, 2 * E):
        part = p_ref[e * S:(e + 1) * S][None]
        if e == E:
            o_ref[0:1, :] = acc
            acc = part
        else:
            acc = acc + part
    o_ref[1:2, :] = acc


def _possum(parts):
    return pl.pallas_call(
        _possum_kernel,
        grid=(1,),
        in_specs=[pl.BlockSpec((2 * E * S,), lambda i: (0,))],
        out_specs=pl.BlockSpec((2, S), lambda i: (0, 0)),
        out_shape=jax.ShapeDtypeStruct((2, S), jnp.int32),
    )(parts)


_CROWS = 8


def _combine_kernel(pos_ref, *refs):
    del pos_ref
    a = refs[:_CROWS]
    bb = refs[_CROWS:2 * _CROWS]
    x_ref, g_ref, b_ref, out_ref = refs[2 * _CROWS:]
    rows = [a[j][0, 0, :] + bb[j][0, 0, :] for j in range(_CROWS)]
    y = x_ref[...] + jnp.concatenate([r[None] for r in rows], axis=0)
    out_ref[...] = _ln_rows(y, g_ref[...], b_ref[...])


def _combine_ln(yg3, pos, x, g, b):
    grid_spec = pltpu.PrefetchScalarGridSpec(
        num_scalar_prefetch=1,
        grid=(S // _CROWS,),
        in_specs=(
            [pl.BlockSpec((1, 1, D), functools.partial(
                lambda j, i, p: (p[0, i * _CROWS + j], 0, 0), j))
             for j in range(_CROWS)]
            + [pl.BlockSpec((1, 1, D), functools.partial(
                lambda j, i, p: (p[1, i * _CROWS + j], 0, 0), j))
               for j in range(_CROWS)]
            + [pl.BlockSpec((_CROWS, D), lambda i, p: (i, 0)),
               pl.BlockSpec((1, D), lambda i, p: (0, 0)),
               pl.BlockSpec((1, D), lambda i, p: (0, 0))]
        ),
        out_specs=pl.BlockSpec((_CROWS, D), lambda i, p: (i, 0)),
    )
    return pl.pallas_call(
        _combine_kernel,
        grid_spec=grid_spec,
        out_shape=jax.ShapeDtypeStruct((S, D), jnp.float32),
    )(pos, *([yg3] * _CROWS), *([yg3] * _CROWS), x,
      g.reshape(1, D), b.reshape(1, D))


def _head_kernel(x_ref, g_ref, b_ref, w_ref, hb_ref, o_ref):
    xb = _ln_rows(x_ref[...], g_ref[...], b_ref[...])
    o_ref[...] = lax.dot_general(
        xb, w_ref[...], (((1,), (1,)), ((), ())),
        preferred_element_type=jnp.float32) + hb_ref[...]


def _head(x, lfg, lfb, hW, hb):
    BM, BN = 512, 1280
    return pl.pallas_call(
        _head_kernel,
        grid=(S // BM, V // BN),
        in_specs=[
            pl.BlockSpec((BM, D), lambda i, j: (i, 0)),
            pl.BlockSpec((1, D), lambda i, j: (0, 0)),
            pl.BlockSpec((1, D), lambda i, j: (0, 0)),
            pl.BlockSpec((BN, D), lambda i, j: (j, 0)),
            pl.BlockSpec((1, BN), lambda i, j: (0, j)),
        ],
        out_specs=pl.BlockSpec((BM, BN), lambda i, j: (i, j)),
        out_shape=jax.ShapeDtypeStruct((S, V), jnp.float32),
    )(x, lfg.reshape(1, D), lfb.reshape(1, D), hW,
      hb.reshape(1, V))


def kernel(input_ids, emb, Wqkv, bqkv, Wo, bo, gW, gb, W1, b1, W2, b2,
           n1g, n1b, n2g, n2b, lfg, lfb, hW, hb):
    ids = input_ids.reshape(S).astype(jnp.int32)
    x = _sc_gather(emb, ids, S)
    for l in range(L):
        qkv = _qkv_proj(x, Wqkv[l], bqkv[l])
        o = _attention(qkv)
        x = _oproj_ln(o, Wo[l], bo[l], x, n1g[l], n1b[l])
        ti, tp = _gate(x, gW[l], gb[l])
        disp, pslot, blk, parts = _route_sc(
            ti.reshape(_NA), tp.reshape(_NA))
        pos = _possum(parts)
        xg = _sc_gather(x, disp, _NSLOT)
        yg = _grouped_moe(xg, W1[l], b1[l], W2[l], b2[l], pslot, blk)
        x = _combine_ln(yg, pos, x, n2g[l], n2b[l])
    out = _head(x, lfg, lfb, hW, hb)
    return out.reshape(B, S, V)
